# baseline (speedup 1.0000x reference)
import functools

import jax
import jax.numpy as jnp
from jax import lax
from jax.experimental import pallas as pl
from jax.experimental.pallas import tpu as pltpu
from jax.experimental.pallas import tpu_sc as plsc

_B, _H, _C, _D, _F = 2, 16, 2048, 128, 128
_G = 512
_BH = _B * _H
_NC = 2
_NS = 16
_L = 16
_NV = _C // _L
_HR = 256


def _sc_body(score_hbm, vf_hbm, fkf_hbm,
             val_out, idx_out, idxg_out, vtop_out, fktop_out,
             key_v, idx_v, idxg_v, rows_a, rows_b,
             gs0, gs1, ws0, ws1):
    wid = lax.axis_index("s") * _NC + lax.axis_index("c")
    pltpu.sync_copy(score_hbm.at[wid], key_v)

    @plsc.parallel_loop(0, _NV, unroll=4)
    def _(v):
        b = v * _L
        k = key_v[pl.ds(b, _L)]
        i = lax.iota(jnp.int32, _L) + b
        ks, vs = plsc.sort_key_val(k, i, descending=True)
        key_v[pl.ds(b, _L)] = ks
        idx_v[pl.ds(b, _L)] = vs

    for lvl in range(7):
        M = 2 ** lvl
        sw = (M + 1) // 2
        nb = _NV // (2 * M)

        @plsc.parallel_loop(0, nb * sw, unroll=min(4, nb * sw))
        def _(t, M=M, sw=sw):
            m_ = t // sw
            u = t % sw
            bstart = m_ * (2 * M) + M
            p1 = (bstart + u) * _L
            p2 = (bstart + M - 1 - u) * _L
            k1 = key_v[pl.ds(p1, _L)]
            i1 = idx_v[pl.ds(p1, _L)]
            k2 = key_v[pl.ds(p2, _L)]
            i2 = idx_v[pl.ds(p2, _L)]
            key_v[pl.ds(p1, _L)] = lax.rev(k2, (0,))
            key_v[pl.ds(p2, _L)] = lax.rev(k1, (0,))
            idx_v[pl.ds(p1, _L)] = lax.rev(i2, (0,))
            idx_v[pl.ds(p2, _L)] = lax.rev(i1, (0,))

        stages = []
        s_ = M
        while s_ >= 1:
            stages.append(s_)
            s_ //= 2
        si = 0
        while si < len(stages):
            if si + 1 < len(stages):
                s1, s2 = stages[si], stages[si + 1]
                b2 = s2.bit_length() - 1

                @plsc.parallel_loop(0, _NV // 4, unroll=4)
                def _(t, s1=s1, s2=s2, b2=b2):
                    x = ((t >> b2) << (b2 + 2)) | (t & (s2 - 1))
                    p0 = x * _L
                    p1 = (x + s2) * _L
                    p2 = (x + s1) * _L
                    p3 = (x + s1 + s2) * _L
                    k0 = key_v[pl.ds(p0, _L)]
                    k1 = key_v[pl.ds(p1, _L)]
                    k2 = key_v[pl.ds(p2, _L)]
                    k3 = key_v[pl.ds(p3, _L)]
                    i0 = idx_v[pl.ds(p0, _L)]
                    i1 = idx_v[pl.ds(p1, _L)]
                    i2 = idx_v[pl.ds(p2, _L)]
                    i3 = idx_v[pl.ds(p3, _L)]
                    m = k0 >= k2
                    k0, k2 = jnp.where(m, k0, k2), jnp.where(m, k2, k0)
                    i0, i2 = jnp.where(m, i0, i2), jnp.where(m, i2, i0)
                    m = k1 >= k3
                    k1, k3 = jnp.where(m, k1, k3), jnp.where(m, k3, k1)
                    i1, i3 = jnp.where(m, i1, i3), jnp.where(m, i3, i1)
                    m = k0 >= k1
                    k0, k1 = jnp.where(m, k0, k1), jnp.where(m, k1, k0)
                    i0, i1 = jnp.where(m, i0, i1), jnp.where(m, i1, i0)
                    m = k2 >= k3
                    k2, k3 = jnp.where(m, k2, k3), jnp.where(m, k3, k2)
                    i2, i3 = jnp.where(m, i2, i3), jnp.where(m, i3, i2)
                    key_v[pl.ds(p0, _L)] = k0
                    key_v[pl.ds(p1, _L)] = k1
                    key_v[pl.ds(p2, _L)] = k2
                    key_v[pl.ds(p3, _L)] = k3
                    idx_v[pl.ds(p0, _L)] = i0
                    idx_v[pl.ds(p1, _L)] = i1
                    idx_v[pl.ds(p2, _L)] = i2
                    idx_v[pl.ds(p3, _L)] = i3
                si += 2
            else:
                sl = stages[si]
                bbit = sl.bit_length() - 1

                @plsc.parallel_loop(0, _NV // 2, unroll=4)
                def _(t, sl=sl, bbit=bbit):
                    p = ((t >> bbit) << (bbit + 1)) | (t & (sl - 1))
                    pa = p * _L
                    pb = (p + sl) * _L
                    ka = key_v[pl.ds(pa, _L)]
                    kb = key_v[pl.ds(pb, _L)]
                    ia = idx_v[pl.ds(pa, _L)]
                    ib = idx_v[pl.ds(pb, _L)]
                    m = ka >= kb
                    key_v[pl.ds(pa, _L)] = jnp.where(m, ka, kb)
                    key_v[pl.ds(pb, _L)] = jnp.where(m, kb, ka)
                    idx_v[pl.ds(pa, _L)] = jnp.where(m, ia, ib)
                    idx_v[pl.ds(pb, _L)] = jnp.where(m, ib, ia)
                si += 1

        @plsc.parallel_loop(0, _NV, unroll=4)
        def _(v):
            b = v * _L
            k = key_v[pl.ds(b, _L)]
            i = idx_v[pl.ds(b, _L)]
            ks, vs = plsc.sort_key_val(k, i, descending=True)
            key_v[pl.ds(b, _L)] = ks
            idx_v[pl.ds(b, _L)] = vs

    lanes2 = lax.iota(jnp.int32, _L) * 2

    def sweep(parity):
        def pair_t(t, acc):
            a = lanes2 + t * (2 * _L) + parity
            asafe = jnp.minimum(a, _C - 2)
            ka = plsc.load_gather(key_v, [asafe])
            kb = plsc.load_gather(key_v, [asafe + 1])
            ia = plsc.load_gather(idx_v, [asafe])
            ib = plsc.load_gather(idx_v, [asafe + 1])
            m = (ka == kb) & (ia > ib) & (a <= _C - 2)
            plsc.store_scatter(idx_v, [asafe], ib, mask=m)
            plsc.store_scatter(idx_v, [asafe + 1], ia, mask=m)
            return acc + jnp.sum(m.astype(jnp.int32))
        return lax.fori_loop(0, _C // (2 * _L), pair_t, 0)

    lax.while_loop(lambda c: c > 0, lambda c: sweep(0) + sweep(1),
                   jnp.int32(1))

    pltpu.sync_copy(key_v.at[pl.ds(0, _G)], val_out.at[wid])
    pltpu.sync_copy(idx_v.at[pl.ds(0, _G)], idx_out.at[wid])

    off = wid * _C
    for t in range(_G // _L):
        loc = idx_v[pl.ds(t * _L, _L)]
        idxg_v[t // 8, pl.ds((t % 8) * _L, _L)] = loc + off

    pltpu.sync_copy(idxg_v, idxg_out.at[wid])

    srcs = (vf_hbm, vf_hbm, fkf_hbm, fkf_hbm)
    dsts = (vtop_out, vtop_out, fktop_out, fktop_out)
    bufs = (rows_a, rows_b)
    gsems = (gs0, gs1)
    wsems = (ws0, ws1)
    nch = 4
    gcps = [None] * nch
    wcps = [None] * nch
    for c in range(nch):
        bi = c % 2
        half = c % 2
        if c >= 2:
            wcps[c - 2].wait()
        gcps[c] = [
            pltpu.async_copy(srcs[c].at[idxg_v.at[2 * half + j]],
                             bufs[bi].at[pl.ds(j * 128, 128)], gsems[bi])
            for j in range(2)
        ]
        if c >= 1:
            pb = c - 1
            for g in gcps[pb]:
                g.wait()
            wcps[pb] = pltpu.async_copy(
                bufs[pb % 2],
                dsts[pb].at[pl.ds(wid * _G + (pb % 2) * _HR, _HR)],
                wsems[pb % 2])
    for g in gcps[nch - 1]:
        g.wait()
    wcps[nch - 1] = pltpu.async_copy(
        bufs[(nch - 1) % 2],
        dsts[nch - 1].at[pl.ds(wid * _G + ((nch - 1) % 2) * _HR, _HR)],
        wsems[(nch - 1) % 2])
    wcps[nch - 2].wait()
    wcps[nch - 1].wait()


def _kgather_body(kf_hbm, idxg_hbm, ktop_out,
                  idxg_v, rows_a, rows_b, gs0, gs1, ws0, ws1):
    wid = lax.axis_index("s") * _NC + lax.axis_index("c")
    pltpu.sync_copy(idxg_hbm.at[wid], idxg_v)
    bufs = (rows_a, rows_b)
    gsems = (gs0, gs1)
    wsems = (ws0, ws1)
    gcps = [None, None]
    for c in range(2):
        gcps[c] = [
            pltpu.async_copy(kf_hbm.at[idxg_v.at[2 * c + j]],
                             bufs[c].at[pl.ds(j * 128, 128)], gsems[c])
            for j in range(2)
        ]
    wcps = [None, None]
    for c in range(2):
        for g in gcps[c]:
            g.wait()
        wcps[c] = pltpu.async_copy(
            bufs[c], ktop_out.at[pl.ds(wid * _G + c * _HR, _HR)], wsems[c])
    for c in range(2):
        wcps[c].wait()


@functools.cache
def _sc_select_call():
  return pl.kernel(
    _sc_body,
    mesh=plsc.VectorSubcoreMesh(core_axis_name="c", subcore_axis_name="s"),
    compiler_params=pltpu.CompilerParams(needs_layout_passes=False),
    out_type=(
        jax.ShapeDtypeStruct((_BH, _G), jnp.float32),
        jax.ShapeDtypeStruct((_BH, _G), jnp.int32),
        jax.ShapeDtypeStruct((_BH, _G // 128, 128), jnp.int32),
        jax.ShapeDtypeStruct((_BH * _G, _D), jnp.float32),
        jax.ShapeDtypeStruct((_BH * _G, _F), jnp.float32),
    ),
    scratch_types=[
        pltpu.VMEM((_C,), jnp.float32),
        pltpu.VMEM((_C,), jnp.int32),
        pltpu.VMEM((_G // 128, 128), jnp.int32),
        pltpu.VMEM((_HR, _D), jnp.float32),
        pltpu.VMEM((_HR, _D), jnp.float32),
        pltpu.SemaphoreType.DMA,
        pltpu.SemaphoreType.DMA,
        pltpu.SemaphoreType.DMA,
        pltpu.SemaphoreType.DMA,
    ],
  )


@functools.cache
def _kgather_call():
  return pl.kernel(
    _kgather_body,
    mesh=plsc.VectorSubcoreMesh(core_axis_name="c", subcore_axis_name="s"),
    compiler_params=pltpu.CompilerParams(needs_layout_passes=False),
    out_type=jax.ShapeDtypeStruct((_BH * _G, _D), jnp.float32),
    scratch_types=[
        pltpu.VMEM((_G // 128, 128), jnp.int32),
        pltpu.VMEM((_HR, _D), jnp.float32),
        pltpu.VMEM((_HR, _D), jnp.float32),
        pltpu.SemaphoreType.DMA,
        pltpu.SemaphoreType.DMA,
        pltpu.SemaphoreType.DMA,
        pltpu.SemaphoreType.DMA,
    ],
  )


_EB = 8


def _einsum_body(fk_ref, v_ref, h_ref, s_ref):
    for j in range(_EB):
        fk = fk_ref[j]
        v = v_ref[j]
        h_ref[j] = lax.dot_general(fk, v, (((0,), (0,)), ((), ())),
                                   preferred_element_type=jnp.float32)
        s_ref[j] = jnp.sum(fk, axis=0, keepdims=True)


def _einsum(fk_top, v_top):
    return pl.pallas_call(
        _einsum_body,
        grid=(_BH // _EB,),
        in_specs=[
            pl.BlockSpec((_EB, _G, _F), lambda i: (i, 0, 0)),
            pl.BlockSpec((_EB, _G, _D), lambda i: (i, 0, 0)),
        ],
        out_specs=[
            pl.BlockSpec((_EB, _F, _D), lambda i: (i, 0, 0)),
            pl.BlockSpec((_EB, 1, _F), lambda i: (i, 0, 0)),
        ],
        out_shape=[
            jax.ShapeDtypeStruct((_BH, _F, _D), jnp.float32),
            jax.ShapeDtypeStruct((_BH, 1, _F), jnp.float32),
        ],
    )(fk_top, v_top)


def kernel(k_c, v_c, fk_c, score_c):
    score2 = score_c.reshape(_BH, _C)
    kf = k_c.reshape(_BH * _C, _D)
    vf = v_c.reshape(_BH * _C, _D)
    fkf = fk_c.reshape(_BH * _C, _F)
    top_val, top_idx, idxg, v_top, fk_top = _sc_select_call()(
        score2, vf, fkf)
    k_top = _kgather_call()(kf, idxg)
    h_sum, s_sum = _einsum(fk_top.reshape(_BH, _G, _F),
                           v_top.reshape(_BH, _G, _D))
    return (
        h_sum.reshape(_B, _H, _F, _D),
        s_sum.reshape(_B, _H, _F),
        top_val.reshape(_B, _H, _G),
        top_idx.reshape(_B, _H, _G),
        k_top.reshape(_B, _H, _G, _D),
    )


# speedup vs baseline: 1.0168x; 1.0168x over previous
import functools

import jax
import jax.numpy as jnp
from jax import lax
from jax.experimental import pallas as pl
from jax.experimental.pallas import tpu as pltpu
from jax.experimental.pallas import tpu_sc as plsc

_B, _H, _C, _D, _F = 2, 16, 2048, 128, 128
_G = 512
_BH = _B * _H
_NC = 2
_NS = 16
_L = 16
_NV = _C // _L
_HR = 256


def _sc_body(score_hbm, vf_hbm, fkf_hbm,
             val_out, idx_out, idxg_out, vtop_out, fktop_out,
             key_v, idx_v, idxg_v, rows_a, rows_b,
             gs0, gs1, ws0, ws1):
    wid = lax.axis_index("s") * _NC + lax.axis_index("c")
    pltpu.sync_copy(score_hbm.at[wid], key_v)

    @plsc.parallel_loop(0, _NV, unroll=4)
    def _(v):
        b = v * _L
        k = key_v[pl.ds(b, _L)]
        i = lax.iota(jnp.int32, _L) + b
        ks, vs = plsc.sort_key_val(k, i, descending=True)
        key_v[pl.ds(b, _L)] = ks
        idx_v[pl.ds(b, _L)] = vs

    for lvl in range(7):
        M = 2 ** lvl
        sw = (M + 1) // 2
        nb = _NV // (2 * M)

        @plsc.parallel_loop(0, nb * sw, unroll=min(4, nb * sw))
        def _(t, M=M, sw=sw):
            m_ = t // sw
            u = t % sw
            bstart = m_ * (2 * M) + M
            p1 = (bstart + u) * _L
            p2 = (bstart + M - 1 - u) * _L
            k1 = key_v[pl.ds(p1, _L)]
            i1 = idx_v[pl.ds(p1, _L)]
            k2 = key_v[pl.ds(p2, _L)]
            i2 = idx_v[pl.ds(p2, _L)]
            key_v[pl.ds(p1, _L)] = lax.rev(k2, (0,))
            key_v[pl.ds(p2, _L)] = lax.rev(k1, (0,))
            idx_v[pl.ds(p1, _L)] = lax.rev(i2, (0,))
            idx_v[pl.ds(p2, _L)] = lax.rev(i1, (0,))

        stages = []
        s_ = M
        while s_ >= 1:
            stages.append(s_)
            s_ //= 2
        si = 0
        while si < len(stages):
            if si + 1 < len(stages):
                s1, s2 = stages[si], stages[si + 1]
                b2 = s2.bit_length() - 1

                @plsc.parallel_loop(0, _NV // 4, unroll=2)
                def _(t, s1=s1, s2=s2, b2=b2):
                    x = ((t >> b2) << (b2 + 2)) | (t & (s2 - 1))
                    p0 = x * _L
                    p1 = (x + s2) * _L
                    p2 = (x + s1) * _L
                    p3 = (x + s1 + s2) * _L
                    k0 = key_v[pl.ds(p0, _L)]
                    k1 = key_v[pl.ds(p1, _L)]
                    k2 = key_v[pl.ds(p2, _L)]
                    k3 = key_v[pl.ds(p3, _L)]
                    i0 = idx_v[pl.ds(p0, _L)]
                    i1 = idx_v[pl.ds(p1, _L)]
                    i2 = idx_v[pl.ds(p2, _L)]
                    i3 = idx_v[pl.ds(p3, _L)]
                    m = k0 >= k2
                    k0, k2 = jnp.where(m, k0, k2), jnp.where(m, k2, k0)
                    i0, i2 = jnp.where(m, i0, i2), jnp.where(m, i2, i0)
                    m = k1 >= k3
                    k1, k3 = jnp.where(m, k1, k3), jnp.where(m, k3, k1)
                    i1, i3 = jnp.where(m, i1, i3), jnp.where(m, i3, i1)
                    m = k0 >= k1
                    k0, k1 = jnp.where(m, k0, k1), jnp.where(m, k1, k0)
                    i0, i1 = jnp.where(m, i0, i1), jnp.where(m, i1, i0)
                    m = k2 >= k3
                    k2, k3 = jnp.where(m, k2, k3), jnp.where(m, k3, k2)
                    i2, i3 = jnp.where(m, i2, i3), jnp.where(m, i3, i2)
                    key_v[pl.ds(p0, _L)] = k0
                    key_v[pl.ds(p1, _L)] = k1
                    key_v[pl.ds(p2, _L)] = k2
                    key_v[pl.ds(p3, _L)] = k3
                    idx_v[pl.ds(p0, _L)] = i0
                    idx_v[pl.ds(p1, _L)] = i1
                    idx_v[pl.ds(p2, _L)] = i2
                    idx_v[pl.ds(p3, _L)] = i3
                si += 2
            else:
                sl = stages[si]
                bbit = sl.bit_length() - 1

                @plsc.parallel_loop(0, _NV // 2, unroll=4)
                def _(t, sl=sl, bbit=bbit):
                    p = ((t >> bbit) << (bbit + 1)) | (t & (sl - 1))
                    pa = p * _L
                    pb = (p + sl) * _L
                    ka = key_v[pl.ds(pa, _L)]
                    kb = key_v[pl.ds(pb, _L)]
                    ia = idx_v[pl.ds(pa, _L)]
                    ib = idx_v[pl.ds(pb, _L)]
                    m = ka >= kb
                    key_v[pl.ds(pa, _L)] = jnp.where(m, ka, kb)
                    key_v[pl.ds(pb, _L)] = jnp.where(m, kb, ka)
                    idx_v[pl.ds(pa, _L)] = jnp.where(m, ia, ib)
                    idx_v[pl.ds(pb, _L)] = jnp.where(m, ib, ia)
                si += 1

        @plsc.parallel_loop(0, _NV, unroll=4)
        def _(v):
            b = v * _L
            k = key_v[pl.ds(b, _L)]
            i = idx_v[pl.ds(b, _L)]
            ks, vs = plsc.sort_key_val(k, i, descending=True)
            key_v[pl.ds(b, _L)] = ks
            idx_v[pl.ds(b, _L)] = vs

    lanes2 = lax.iota(jnp.int32, _L) * 2

    def sweep(parity):
        def pair_t(t, acc):
            a = lanes2 + t * (2 * _L) + parity
            asafe = jnp.minimum(a, _C - 2)
            ka = plsc.load_gather(key_v, [asafe])
            kb = plsc.load_gather(key_v, [asafe + 1])
            ia = plsc.load_gather(idx_v, [asafe])
            ib = plsc.load_gather(idx_v, [asafe + 1])
            m = (ka == kb) & (ia > ib) & (a <= _C - 2)
            plsc.store_scatter(idx_v, [asafe], ib, mask=m)
            plsc.store_scatter(idx_v, [asafe + 1], ia, mask=m)
            return acc + jnp.sum(m.astype(jnp.int32))
        return lax.fori_loop(0, _C // (2 * _L), pair_t, 0)

    lax.while_loop(lambda c: c > 0, lambda c: sweep(0) + sweep(1),
                   jnp.int32(1))

    pltpu.sync_copy(key_v.at[pl.ds(0, _G)], val_out.at[wid])
    pltpu.sync_copy(idx_v.at[pl.ds(0, _G)], idx_out.at[wid])

    off = wid * _C
    for t in range(_G // _L):
        loc = idx_v[pl.ds(t * _L, _L)]
        idxg_v[t // 8, pl.ds((t % 8) * _L, _L)] = loc + off

    pltpu.sync_copy(idxg_v, idxg_out.at[wid])

    srcs = (vf_hbm, vf_hbm, fkf_hbm, fkf_hbm)
    dsts = (vtop_out, vtop_out, fktop_out, fktop_out)
    bufs = (rows_a, rows_b)
    gsems = (gs0, gs1)
    wsems = (ws0, ws1)
    nch = 4
    gcps = [None] * nch
    wcps = [None] * nch
    for c in range(nch):
        bi = c % 2
        half = c % 2
        if c >= 2:
            wcps[c - 2].wait()
        gcps[c] = [
            pltpu.async_copy(srcs[c].at[idxg_v.at[2 * half + j]],
                             bufs[bi].at[pl.ds(j * 128, 128)], gsems[bi])
            for j in range(2)
        ]
        if c >= 1:
            pb = c - 1
            for g in gcps[pb]:
                g.wait()
            wcps[pb] = pltpu.async_copy(
                bufs[pb % 2],
                dsts[pb].at[pl.ds(wid * _G + (pb % 2) * _HR, _HR)],
                wsems[pb % 2])
    for g in gcps[nch - 1]:
        g.wait()
    wcps[nch - 1] = pltpu.async_copy(
        bufs[(nch - 1) % 2],
        dsts[nch - 1].at[pl.ds(wid * _G + ((nch - 1) % 2) * _HR, _HR)],
        wsems[(nch - 1) % 2])
    wcps[nch - 2].wait()
    wcps[nch - 1].wait()


def _kgather_body(kf_hbm, idxg_hbm, ktop_out,
                  idxg_v, rows_a, rows_b, gs0, gs1, ws0, ws1):
    wid = lax.axis_index("s") * _NC + lax.axis_index("c")
    pltpu.sync_copy(idxg_hbm.at[wid], idxg_v)
    bufs = (rows_a, rows_b)
    gsems = (gs0, gs1)
    wsems = (ws0, ws1)
    gcps = [None, None]
    for c in range(2):
        gcps[c] = [
            pltpu.async_copy(kf_hbm.at[idxg_v.at[2 * c + j]],
                             bufs[c].at[pl.ds(j * 128, 128)], gsems[c])
            for j in range(2)
        ]
    wcps = [None, None]
    for c in range(2):
        for g in gcps[c]:
            g.wait()
        wcps[c] = pltpu.async_copy(
            bufs[c], ktop_out.at[pl.ds(wid * _G + c * _HR, _HR)], wsems[c])
    for c in range(2):
        wcps[c].wait()


@functools.cache
def _sc_select_call():
  return pl.kernel(
    _sc_body,
    mesh=plsc.VectorSubcoreMesh(core_axis_name="c", subcore_axis_name="s"),
    compiler_params=pltpu.CompilerParams(needs_layout_passes=False),
    out_type=(
        jax.ShapeDtypeStruct((_BH, _G), jnp.float32),
        jax.ShapeDtypeStruct((_BH, _G), jnp.int32),
        jax.ShapeDtypeStruct((_BH, _G // 128, 128), jnp.int32),
        jax.ShapeDtypeStruct((_BH * _G, _D), jnp.float32),
        jax.ShapeDtypeStruct((_BH * _G, _F), jnp.float32),
    ),
    scratch_types=[
        pltpu.VMEM((_C,), jnp.float32),
        pltpu.VMEM((_C,), jnp.int32),
        pltpu.VMEM((_G // 128, 128), jnp.int32),
        pltpu.VMEM((_HR, _D), jnp.float32),
        pltpu.VMEM((_HR, _D), jnp.float32),
        pltpu.SemaphoreType.DMA,
        pltpu.SemaphoreType.DMA,
        pltpu.SemaphoreType.DMA,
        pltpu.SemaphoreType.DMA,
    ],
  )


@functools.cache
def _kgather_call():
  return pl.kernel(
    _kgather_body,
    mesh=plsc.VectorSubcoreMesh(core_axis_name="c", subcore_axis_name="s"),
    compiler_params=pltpu.CompilerParams(needs_layout_passes=False),
    out_type=jax.ShapeDtypeStruct((_BH * _G, _D), jnp.float32),
    scratch_types=[
        pltpu.VMEM((_G // 128, 128), jnp.int32),
        pltpu.VMEM((_HR, _D), jnp.float32),
        pltpu.VMEM((_HR, _D), jnp.float32),
        pltpu.SemaphoreType.DMA,
        pltpu.SemaphoreType.DMA,
        pltpu.SemaphoreType.DMA,
        pltpu.SemaphoreType.DMA,
    ],
  )


_EB = 8


def _einsum_body(fk_ref, v_ref, h_ref, s_ref):
    for j in range(_EB):
        fk = fk_ref[j]
        v = v_ref[j]
        h_ref[j] = lax.dot_general(fk, v, (((0,), (0,)), ((), ())),
                                   preferred_element_type=jnp.float32)
        s_ref[j] = jnp.sum(fk, axis=0, keepdims=True)


def _einsum(fk_top, v_top):
    return pl.pallas_call(
        _einsum_body,
        grid=(_BH // _EB,),
        in_specs=[
            pl.BlockSpec((_EB, _G, _F), lambda i: (i, 0, 0)),
            pl.BlockSpec((_EB, _G, _D), lambda i: (i, 0, 0)),
        ],
        out_specs=[
            pl.BlockSpec((_EB, _F, _D), lambda i: (i, 0, 0)),
            pl.BlockSpec((_EB, 1, _F), lambda i: (i, 0, 0)),
        ],
        out_shape=[
            jax.ShapeDtypeStruct((_BH, _F, _D), jnp.float32),
            jax.ShapeDtypeStruct((_BH, 1, _F), jnp.float32),
        ],
    )(fk_top, v_top)


def kernel(k_c, v_c, fk_c, score_c):
    score2 = score_c.reshape(_BH, _C)
    kf = k_c.reshape(_BH * _C, _D)
    vf = v_c.reshape(_BH * _C, _D)
    fkf = fk_c.reshape(_BH * _C, _F)
    top_val, top_idx, idxg, v_top, fk_top = _sc_select_call()(
        score2, vf, fkf)
    k_top = _kgather_call()(kf, idxg)
    h_sum, s_sum = _einsum(fk_top.reshape(_BH, _G, _F),
                           v_top.reshape(_BH, _G, _D))
    return (
        h_sum.reshape(_B, _H, _F, _D),
        s_sum.reshape(_B, _H, _F),
        top_val.reshape(_B, _H, _G),
        top_idx.reshape(_B, _H, _G),
        k_top.reshape(_B, _H, _G, _D),
    )
